# Initial kernel scaffold; baseline (speedup 1.0000x reference)
import jax
import jax.numpy as jnp
from jax.experimental import pallas as pl


def kernel(x, embed, phi_w, phi_b):
    raise NotImplementedError("write your pallas kernel here")


# speedup vs baseline: 2.7254x; 2.7254x over previous
import functools

import numpy as np
import jax
import jax.numpy as jnp
from jax.experimental import pallas as pl

_SCALES = (32, 64, 128, 256, 512, 1024, 2048)
_T = 2048
_C = 512
_K = 1024
_NPHI = 4
_HI = jax.lax.Precision.HIGHEST


def _resize_mat(in_size, out_size):
    scale = out_size / in_size
    inv = 1.0 / scale
    kscale = max(inv, 1.0)
    sample_f = (np.arange(out_size) + 0.5) * inv - 0.5
    x = np.abs(sample_f[None, :] - np.arange(in_size)[:, None]) / kscale
    w = np.maximum(0.0, 1.0 - x)
    tot = w.sum(axis=0, keepdims=True)
    w = np.where(np.abs(tot) > 1e-8, w / tot, 0.0)
    return w.astype(np.float32)


def _k_for_scale(si):
    ticks = np.linspace(1.0 / 3.0 / _NPHI, 1.0 - 1.0 / 3.0 / _NPHI, _NPHI)
    return int(np.argmin(np.abs(ticks - si / (len(_SCALES) - 1))))


def _dot(a, b, precision=None):
    return jax.lax.dot_general(a, b, (((1,), (0,)), ((), ())),
                               precision=precision,
                               preferred_element_type=jnp.float32)


def _scale_body(*refs, pn, last):
    if last:
        if pn != _T:
            fr_ref, rt_ref, e_ref, ebt_ref, wtb_ref, b_ref, xt_ref, out_ref, idx_ref = refs
        else:
            fr_ref, e_ref, ebt_ref, wtb_ref, b_ref, xt_ref, out_ref, idx_ref = refs
    else:
        if pn != _T:
            fr_ref, rt_ref, e_ref, ebt_ref, wtb_ref, b_ref, out_ref, idx_ref = refs
        else:
            fr_ref, e_ref, ebt_ref, wtb_ref, b_ref, out_ref, idx_ref = refs

    fr = fr_ref[0]
    if pn != _T:
        z = _dot(rt_ref[...], fr, precision=_HI)
    else:
        z = fr
    e = e_ref[...]
    e2 = jnp.sum(e * e, axis=1)
    zb = z.astype(jnp.bfloat16)
    ebt = ebt_ref[...]

    cs = min(pn, 512)
    idx_parts, h_parts = [], []
    for j0 in range(0, pn, cs):
        zc = zb[j0:j0 + cs]
        prod = _dot(zc, ebt)
        zf = z[j0:j0 + cs]
        z2 = jnp.sum(zf * zf, axis=1)
        d = z2[:, None] - 2.0 * prod + e2[None, :]
        m = jnp.min(d, axis=1)
        iota = jax.lax.broadcasted_iota(jnp.int32, (cs, _K), 1)
        idxc = jnp.min(jnp.where(d == m[:, None], iota, _K), axis=1)
        idx_parts.append(idxc)
        oh = (iota == idxc[:, None]).astype(jnp.float32)
        h_parts.append(_dot(oh, e, precision=_HI))
    idx_ref[0, 0] = (jnp.concatenate(idx_parts)
                     if len(idx_parts) > 1 else idx_parts[0])
    h = jnp.concatenate(h_parts, axis=0) if len(h_parts) > 1 else h_parts[0]

    f = _T // pn
    if f == 1:
        hu = h
    else:
        hm = jnp.concatenate([h[:1], h[:-1]], axis=0)
        hp = jnp.concatenate([h[1:], h[-1:]], axis=0)
        cols = []
        for p in range(f):
            off = (p + 0.5) / f - 0.5
            if off < 0:
                cols.append((-off) * hm + (1.0 + off) * h)
            else:
                cols.append((1.0 - off) * h + off * hp)
        hu = jnp.stack(cols, axis=1).reshape(_T, _C)

    hub = hu.astype(jnp.bfloat16)
    wtb = wtb_ref[...]
    bias = b_ref[...]
    zrow = jnp.zeros((1, _C), jnp.bfloat16)
    cw = 512
    for t0 in range(0, _T, cw):
        top = zrow if t0 == 0 else hub[t0 - 1:t0]
        bot = zrow if t0 + cw == _T else hub[t0 + cw:t0 + cw + 1]
        ext = jnp.concatenate([top, hub[t0:t0 + cw], bot], axis=0)
        a0 = _dot(ext, wtb[0])
        a1 = _dot(ext, wtb[1])
        a2 = _dot(ext, wtb[2])
        conv = a0[0:cw] + a1[1:cw + 1] + a2[2:cw + 2]
        h_out = 0.5 * hu[t0:t0 + cw] + 0.5 * (conv + bias)
        res = fr[t0:t0 + cw] - h_out
        if last:
            out_ref[0, t0:t0 + cw] = xt_ref[0, t0:t0 + cw] - res
        else:
            out_ref[0, t0:t0 + cw] = res


def _scale_call(frest, xt, consts, pn, last):
    B = frest.shape[0]
    rt, e, ebt, wtb, bias = consts
    inputs = [frest]
    in_specs = [pl.BlockSpec((1, _T, _C), lambda b: (b, 0, 0))]
    if pn != _T:
        inputs.append(rt)
        in_specs.append(pl.BlockSpec((pn, _T), lambda b: (0, 0)))
    inputs += [e, ebt, wtb, bias]
    in_specs += [
        pl.BlockSpec((_K, _C), lambda b: (0, 0)),
        pl.BlockSpec((_C, _K), lambda b: (0, 0)),
        pl.BlockSpec((3, _C, _C), lambda b: (0, 0, 0)),
        pl.BlockSpec((1, _C), lambda b: (0, 0)),
    ]
    if last:
        inputs.append(xt)
        in_specs.append(pl.BlockSpec((1, _T, _C), lambda b: (b, 0, 0)))
    out_specs = [
        pl.BlockSpec((1, _T, _C), lambda b: (b, 0, 0)),
        pl.BlockSpec((1, 1, pn), lambda b: (b, 0, 0)),
    ]
    out_shape = [
        jax.ShapeDtypeStruct((B, _T, _C), jnp.float32),
        jax.ShapeDtypeStruct((B, 1, pn), jnp.int32),
    ]
    body = functools.partial(_scale_body, pn=pn, last=last)
    return pl.pallas_call(
        body,
        grid=(B,),
        in_specs=in_specs,
        out_specs=out_specs,
        out_shape=out_shape,
    )(*inputs)


def kernel(x, embed, phi_w, phi_b):
    B, C, T = x.shape
    xt = x.transpose(0, 2, 1)
    ebt = embed.astype(jnp.bfloat16).T
    frest = xt
    idx_last = None
    for si, pn in enumerate(_SCALES):
        k = _k_for_scale(si)
        wtb = jnp.stack(
            [phi_w[k, :, :, kk].T for kk in range(3)]).astype(jnp.bfloat16)
        bias = phi_b[k].reshape(1, C)
        rt = (jnp.asarray(_resize_mat(T, pn).T) if pn != T else None)
        last = si == len(_SCALES) - 1
        out, idx = _scale_call(frest, xt, (rt, embed, ebt, wtb, bias), pn, last)
        frest = out
        if last:
            idx_last = idx
    f_hat = frest.transpose(0, 2, 1)
    return f_hat, idx_last.reshape(B, T)
